# baseline (speedup 1.0000x reference)
import functools

import jax
import jax.numpy as jnp
from jax.experimental import pallas as pl


def _seg_head_kernel(x_ref, w1_ref, b1_ref, w2_ref, b2_ref, w3_ref, b3_ref,
                     out_ref):
    x = x_ref[...]
    h = jnp.dot(x, w1_ref[...], preferred_element_type=jnp.float32, precision=jax.lax.Precision.HIGHEST) + b1_ref[...]
    h = jnp.where(h > 0, h, 0.2 * h)
    h = jnp.dot(h, w2_ref[...], preferred_element_type=jnp.float32, precision=jax.lax.Precision.HIGHEST) + b2_ref[...]
    h = jnp.where(h > 0, h, 0.2 * h)
    out_ref[...] = (
        jnp.dot(h, w3_ref[...], preferred_element_type=jnp.float32, precision=jax.lax.Precision.HIGHEST) + b3_ref[...]
    )


def _seg_head(x, params):
    (w1, b1), (w2, b2), (w3, b3) = params
    b, n, c = x.shape
    l = w3.shape[1]
    tile = 1024
    grid = (b, n // tile)
    out = pl.pallas_call(
        _seg_head_kernel,
        grid=grid,
        in_specs=[
            pl.BlockSpec((1, tile, c), lambda i, j: (i, j, 0)),
            pl.BlockSpec((c, w1.shape[1]), lambda i, j: (0, 0)),
            pl.BlockSpec((w1.shape[1],), lambda i, j: (0,)),
            pl.BlockSpec((w1.shape[1], w2.shape[1]), lambda i, j: (0, 0)),
            pl.BlockSpec((w2.shape[1],), lambda i, j: (0,)),
            pl.BlockSpec((w2.shape[1], l), lambda i, j: (0, 0)),
            pl.BlockSpec((l,), lambda i, j: (0,)),
        ],
        out_specs=pl.BlockSpec((1, tile, l), lambda i, j: (i, j, 0)),
        out_shape=jax.ShapeDtypeStruct((b, n, l), jnp.float32),
    )(x, w1, b1, w2, b2, w3, b3)
    return out


def _sqdist(a, b):
    return jnp.sum((a[:, :, None, :] - b[:, None, :, :]) ** 2, axis=-1)


def _fps_kernel(S, x_ref, y_ref, z_ref, out_ref):
    b, n = x_ref.shape
    x = x_ref[...]
    y = y_ref[...]
    z = z_ref[...]
    iota_n = jax.lax.broadcasted_iota(jnp.int32, (b, n), 1)
    iota_s = jax.lax.broadcasted_iota(jnp.int32, (b, S), 1)

    def body(i, state):
        dists, sel, idx_col = state
        onehot = (iota_n == idx_col).astype(jnp.float32)
        lx = jnp.sum(x * onehot, axis=1, keepdims=True)
        ly = jnp.sum(y * onehot, axis=1, keepdims=True)
        lz = jnp.sum(z * onehot, axis=1, keepdims=True)
        d = (x - lx) ** 2 + (y - ly) ** 2 + (z - lz) ** 2
        dists = jnp.minimum(dists, d)
        m = jnp.max(dists, axis=1, keepdims=True)
        idx_col = jnp.min(jnp.where(dists == m, iota_n, n), axis=1,
                          keepdims=True)
        sel = jnp.where(iota_s == i, idx_col, sel)
        return (dists, sel, idx_col)

    dists0 = jnp.full((b, n), 1e10, dtype=jnp.float32)
    sel0 = jnp.zeros((b, S), dtype=jnp.int32)
    idx0 = jnp.zeros((b, 1), dtype=jnp.int32)
    _, sel, _ = jax.lax.fori_loop(1, S, body, (dists0, sel0, idx0))
    out_ref[...] = sel


def _fps(xyz, S):
    b, n, _ = xyz.shape
    x = xyz[..., 0]
    y = xyz[..., 1]
    z = xyz[..., 2]
    return pl.pallas_call(
        functools.partial(_fps_kernel, S),
        in_specs=[pl.BlockSpec((b, n), lambda: (0, 0))] * 3,
        out_specs=pl.BlockSpec((b, S), lambda: (0, 0)),
        out_shape=jax.ShapeDtypeStruct((b, S), jnp.int32),
    )(x, y, z)


def _gather(pts, idx):
    return jax.vmap(lambda p, i: p[i])(pts, idx)


def _ball_query(new_xyz, xyz, radius, K):
    n = xyz.shape[1]
    d2 = _sqdist(new_xyz, xyz)
    nn = jnp.argmin(d2, axis=-1).astype(jnp.int32)
    cand = jnp.where(d2 < radius * radius,
                     jnp.arange(n, dtype=jnp.int32)[None, None, :], n)
    cand = jnp.sort(cand, axis=-1)[..., :K]
    first = cand[..., :1]
    cand = jnp.where(cand == n, jnp.broadcast_to(first, cand.shape), cand)
    cand = jnp.where(cand == n, nn[..., None], cand)
    return cand


def _mlp(x, params):
    for W, b in params:
        x = jnp.maximum(x @ W + b, 0.0)
    return x


def _set_abstraction(xyz, feats, S, radius, K, params):
    idx = _fps(xyz, S)
    new_xyz = _gather(xyz, idx)
    group_idx = _ball_query(new_xyz, xyz, radius, K)
    g_xyz = _gather(xyz, group_idx) - new_xyz[:, :, None, :]
    g_feat = _gather(feats, group_idx)
    x = jnp.concatenate([g_xyz, g_feat], axis=-1)
    x = _mlp(x, params)
    return new_xyz, jnp.max(x, axis=2)


def _set_abstraction_all(xyz, feats, params):
    x = jnp.concatenate([xyz, feats], axis=-1)[:, None, :, :]
    x = _mlp(x, params)
    return jnp.mean(xyz, axis=1, keepdims=True), jnp.max(x, axis=2)


def _fp_kernel(n_layers, with_seg, x1_ref, qt_ref, f1_ref, f2_ref, *refs):
    nw = 3 + (n_layers - 1) * 2 + (6 if with_seg else 0)
    wrefs, outs = refs[:nw], refs[nw:]
    P = x1_ref[0]
    QT = qt_ref[0]
    f1 = f1_ref[0]
    f2 = f2_ref[0]
    R, s = P.shape[0], QT.shape[1]
    d2 = ((P[:, 0:1] - QT[0:1, :]) ** 2
          + (P[:, 1:2] - QT[1:2, :]) ** 2
          + (P[:, 2:3] - QT[2:3, :]) ** 2)
    iota = jax.lax.broadcasted_iota(jnp.int32, (R, s), 1)
    wmat = jnp.zeros((R, s), jnp.float32)
    d2w = d2
    for _ in range(3):
        m = jnp.min(d2w, axis=1, keepdims=True)
        idx = jnp.min(jnp.where(d2w == m, iota, s), axis=1, keepdims=True)
        onehot = iota == idx
        wmat = wmat + jnp.where(onehot, 1.0 / (m + 1e-8), 0.0)
        d2w = jnp.where(onehot, jnp.float32(3.4e38), d2w)
    wsum = jnp.sum(wmat, axis=1, keepdims=True)
    interp = jnp.dot(wmat, f2, preferred_element_type=jnp.float32, precision=jax.lax.Precision.HIGHEST) / wsum
    w1a, w1b, b1 = wrefs[0][...], wrefs[1][...], wrefs[2][...]
    h = (jnp.dot(f1, w1a, preferred_element_type=jnp.float32, precision=jax.lax.Precision.HIGHEST)
         + jnp.dot(interp, w1b, preferred_element_type=jnp.float32, precision=jax.lax.Precision.HIGHEST) + b1)
    h = jnp.maximum(h, 0.0)
    k = 3
    for _ in range(n_layers - 1):
        w, b = wrefs[k][...], wrefs[k + 1][...]
        k += 2
        h = jnp.maximum(
            jnp.dot(h, w, preferred_element_type=jnp.float32, precision=jax.lax.Precision.HIGHEST) + b, 0.0)
    outs[0][0] = h
    if with_seg:
        sw1, sb1, sw2, sb2, sw3, sb3 = (r[...] for r in wrefs[k:k + 6])
        g = jnp.dot(h, sw1, preferred_element_type=jnp.float32, precision=jax.lax.Precision.HIGHEST) + sb1
        g = jnp.where(g > 0, g, 0.2 * g)
        g = jnp.dot(g, sw2, preferred_element_type=jnp.float32, precision=jax.lax.Precision.HIGHEST) + sb2
        g = jnp.where(g > 0, g, 0.2 * g)
        outs[1][0] = jnp.dot(g, sw3, preferred_element_type=jnp.float32, precision=jax.lax.Precision.HIGHEST) + sb3


def _pad_last(a, to):
    c = a.shape[-1]
    if c == to:
        return a
    return jnp.pad(a, [(0, 0)] * (a.ndim - 1) + [(0, to - c)])


def _feature_propagation_fused(xyz1, xyz2, feats1, feats2, params, tile,
                               seg_params=None):
    b, n = xyz1.shape[0], xyz1.shape[1]
    s = xyz2.shape[1]
    c1 = feats1.shape[-1]
    c1p = 16 if c1 < 16 else c1
    c2 = feats2.shape[-1]
    x1p = _pad_last(xyz1, 8)
    qt = jnp.swapaxes(_pad_last(xyz2, 8), 1, 2)
    f1p = _pad_last(feats1, c1p)
    w1 = params[0][0]
    w1a, w1b = w1[:c1], w1[c1:]
    w1a = jnp.pad(w1a, [(0, c1p - c1), (0, 0)])
    weights = [w1a, w1b, params[0][1]]
    for w, bias in params[1:]:
        weights += [w, bias]
    n_layers = len(params)
    with_seg = seg_params is not None
    if with_seg:
        for w, bias in seg_params:
            weights += [w, bias]
    cout = params[-1][0].shape[1]
    grid = (b, n // tile)
    in_specs = [
        pl.BlockSpec((1, tile, 8), lambda i, j: (i, j, 0)),
        pl.BlockSpec((1, 8, s), lambda i, j: (i, 0, 0)),
        pl.BlockSpec((1, tile, c1p), lambda i, j: (i, j, 0)),
        pl.BlockSpec((1, s, c2), lambda i, j: (i, 0, 0)),
    ]
    for wgt in weights:
        if wgt.ndim == 2:
            in_specs.append(pl.BlockSpec(wgt.shape, lambda i, j: (0, 0)))
        else:
            in_specs.append(pl.BlockSpec(wgt.shape, lambda i, j: (0,)))
    out_specs = [pl.BlockSpec((1, tile, cout), lambda i, j: (i, j, 0))]
    out_shape = [jax.ShapeDtypeStruct((b, n, cout), jnp.float32)]
    if with_seg:
        l = seg_params[-1][0].shape[1]
        out_specs.append(pl.BlockSpec((1, tile, l), lambda i, j: (i, j, 0)))
        out_shape.append(jax.ShapeDtypeStruct((b, n, l), jnp.float32))
    outs = pl.pallas_call(
        functools.partial(_fp_kernel, n_layers, with_seg),
        grid=grid,
        in_specs=in_specs,
        out_specs=out_specs,
        out_shape=out_shape,
    )(x1p, qt, f1p, feats2, *weights)
    return outs if with_seg else outs[0]


def _fp3_kernel(f3_ref, f4_ref, w1a_ref, w1b_ref, b1_ref, w2_ref, b2_ref,
                out_ref):
    f3 = f3_ref[0]
    f4 = f4_ref[0]
    h = (jnp.dot(f3, w1a_ref[...], preferred_element_type=jnp.float32, precision=jax.lax.Precision.HIGHEST)
         + jnp.dot(f4, w1b_ref[...], preferred_element_type=jnp.float32, precision=jax.lax.Precision.HIGHEST)
         + b1_ref[...])
    h = jnp.maximum(h, 0.0)
    h = jnp.dot(h, w2_ref[...], preferred_element_type=jnp.float32, precision=jax.lax.Precision.HIGHEST) + b2_ref[...]
    out_ref[0] = jnp.maximum(h, 0.0)


def _feature_propagation_bcast(feats1, feats2, params):
    b, n, c1 = feats1.shape
    c2 = feats2.shape[-1]
    (w1, b1), (w2, b2) = params
    w1a, w1b = w1[:c1], w1[c1:]
    cout = w2.shape[1]
    return pl.pallas_call(
        _fp3_kernel,
        grid=(b,),
        in_specs=[
            pl.BlockSpec((1, n, c1), lambda i: (i, 0, 0)),
            pl.BlockSpec((1, 1, c2), lambda i: (i, 0, 0)),
            pl.BlockSpec(w1a.shape, lambda i: (0, 0)),
            pl.BlockSpec(w1b.shape, lambda i: (0, 0)),
            pl.BlockSpec(b1.shape, lambda i: (0,)),
            pl.BlockSpec(w2.shape, lambda i: (0, 0)),
            pl.BlockSpec(b2.shape, lambda i: (0,)),
        ],
        out_specs=pl.BlockSpec((1, n, cout), lambda i: (i, 0, 0)),
        out_shape=jax.ShapeDtypeStruct((b, n, cout), jnp.float32),
    )(feats1, feats2.reshape(b, 1, c2), w1a, w1b, b1, w2, b2)


def kernel(pointcloud, params):
    xyz0, f0 = pointcloud[..., :3], pointcloud[..., 3:]
    xyz1, f1 = _set_abstraction(xyz0, f0, 1024, 0.1, 32, params['sa1'])
    xyz2, f2 = _set_abstraction(xyz1, f1, 256, 0.2, 64, params['sa2'])
    xyz3, f3 = _set_abstraction(xyz2, f2, 64, 0.4, 128, params['sa3'])
    xyz4, f4 = _set_abstraction_all(xyz3, f3, params['sa_all'])
    f3 = _feature_propagation_bcast(f3, f4, params['fp3'])
    f2 = _feature_propagation_fused(xyz2, xyz3, f2, f3, params['fp2'], 256)
    f1 = _feature_propagation_fused(xyz1, xyz2, f1, f2, params['fp1'], 512)
    point_features, logits = _feature_propagation_fused(
        xyz0, xyz1, pointcloud, f1, params['fp0'], 512,
        seg_params=params['seg'])
    global_features = f4.reshape(f4.shape[0], 512)
    return (point_features, global_features, logits)


# speedup vs baseline: 8.0432x; 8.0432x over previous
import functools

import jax
import jax.numpy as jnp
from jax.experimental import pallas as pl
from jax.experimental.pallas import tpu as pltpu
from jax.experimental.pallas import tpu_sc as plsc

_HI = jax.lax.Precision.HIGHEST


def _dot_ref(x, w):
    return jnp.dot(x.astype(jnp.bfloat16), w.astype(jnp.bfloat16),
                   preferred_element_type=jnp.float32)


def _sqdist(a, b):
    return jnp.sum((a[:, :, None, :] - b[:, None, :, :]) ** 2, axis=-1)


def _fps_kernel(S, x_ref, y_ref, z_ref, out_ref):
    b, n = x_ref.shape
    x = x_ref[...]
    y = y_ref[...]
    z = z_ref[...]
    iota_n = jax.lax.broadcasted_iota(jnp.int32, (b, n), 1)
    iota_s = jax.lax.broadcasted_iota(jnp.int32, (b, S), 1)

    def body(i, state):
        dists, sel, idx_col = state
        onehot = (iota_n == idx_col).astype(jnp.float32)
        lx = jnp.sum(x * onehot, axis=1, keepdims=True)
        ly = jnp.sum(y * onehot, axis=1, keepdims=True)
        lz = jnp.sum(z * onehot, axis=1, keepdims=True)
        d = (x - lx) ** 2 + (y - ly) ** 2 + (z - lz) ** 2
        dists = jnp.minimum(dists, d)
        m = jnp.max(dists, axis=1, keepdims=True)
        idx_col = jnp.min(jnp.where(dists == m, iota_n, n), axis=1,
                          keepdims=True)
        sel = jnp.where(iota_s == i, idx_col, sel)
        return (dists, sel, idx_col)

    dists0 = jnp.full((b, n), 1e10, dtype=jnp.float32)
    sel0 = jnp.zeros((b, S), dtype=jnp.int32)
    idx0 = jnp.zeros((b, 1), dtype=jnp.int32)
    _, sel, _ = jax.lax.fori_loop(1, S, body, (dists0, sel0, idx0))
    out_ref[...] = sel


def _fps(xyz, S):
    b, n, _ = xyz.shape
    x = xyz[..., 0]
    y = xyz[..., 1]
    z = xyz[..., 2]
    return pl.pallas_call(
        functools.partial(_fps_kernel, S),
        in_specs=[pl.BlockSpec((b, n), lambda: (0, 0))] * 3,
        out_specs=pl.BlockSpec((b, S), lambda: (0, 0)),
        out_shape=jax.ShapeDtypeStruct((b, S), jnp.int32),
    )(x, y, z)


def _gather(pts, idx):
    return jax.vmap(lambda p, i: p[i])(pts, idx)


def _ball_query(new_xyz, xyz, radius, K):
    n = xyz.shape[1]
    d2 = _sqdist(new_xyz, xyz)
    nn = jnp.argmin(d2, axis=-1).astype(jnp.int32)
    cand = jnp.where(d2 < radius * radius,
                     jnp.arange(n, dtype=jnp.int32)[None, None, :], n)
    cand = jnp.sort(cand, axis=-1)[..., :K]
    first = cand[..., :1]
    cand = jnp.where(cand == n, jnp.broadcast_to(first, cand.shape), cand)
    cand = jnp.where(cand == n, nn[..., None], cand)
    return cand


def _mlp(x, params):
    for W, b in params:
        x = jnp.maximum(x @ W + b, 0.0)
    return x


def _ballq_kernel(radius2, K, p_ref, c_ref, out_ref):
    pts = p_ref[0]
    ctr = c_ref[0]
    n = pts.shape[1]
    s = ctr.shape[0]
    nchunks = n // 128
    cx, cy, cz = ctr[:, 0:1], ctr[:, 1:2], ctr[:, 2:3]
    iota128 = jax.lax.broadcasted_iota(jnp.int32, (s, 128), 1)
    iota_k = jax.lax.broadcasted_iota(jnp.int32, (s, K), 1)
    folded = jnp.zeros((s, 128), jnp.int32)
    dmin = jnp.full((s, 1), 1e30, jnp.float32)
    nn = jnp.zeros((s, 1), jnp.int32)
    for q in range(nchunks):
        px = pts[0:1, q * 128:(q + 1) * 128]
        py = pts[1:2, q * 128:(q + 1) * 128]
        pz = pts[2:3, q * 128:(q + 1) * 128]
        d = (cx - px) ** 2 + (cy - py) ** 2 + (cz - pz) ** 2
        kept = d < radius2
        bit = jnp.int32(-2**31) if q == 31 else jnp.int32(1 << q)
        folded = folded | jnp.where(kept, bit, 0)
        cm = jnp.min(d, axis=1, keepdims=True)
        carg = jnp.min(jnp.where(d == cm, iota128 + q * 128, n), axis=1,
                       keepdims=True)
        nn = jnp.where(cm < dmin, carg, nn)
        dmin = jnp.minimum(dmin, cm)

    def body(k, state):
        folded, sel = state
        lsb = folded & (-folded)
        f = lsb.astype(jnp.float32)
        q = ((jax.lax.bitcast_convert_type(f, jnp.int32) >> 23) & 255) - 127
        jl = jnp.where(folded != 0, q * 128 + iota128, n)
        m = jnp.min(jl, axis=1, keepdims=True)
        sel = jnp.where(iota_k == k, m, sel)
        folded = jnp.where((iota128 == (m & 127)) & (m < n),
                           folded & (folded - 1), folded)
        return (folded, sel)

    sel0 = jnp.full((s, K), n, jnp.int32)
    _, sel = jax.lax.fori_loop(0, K, body, (folded, sel0))
    first = sel[:, 0:1]
    sel = jnp.where(sel == n, first, sel)
    sel = jnp.where(sel == n, nn, sel)
    out_ref[0] = sel + pl.program_id(0) * n


def _ball_query_global(new_xyzp, xyzt, radius, K):
    b, s, _ = new_xyzp.shape
    n = xyzt.shape[2]
    return pl.pallas_call(
        functools.partial(_ballq_kernel, radius * radius, K),
        grid=(b,),
        in_specs=[
            pl.BlockSpec((1, 8, n), lambda i: (i, 0, 0)),
            pl.BlockSpec((1, s, 8), lambda i: (i, 0, 0)),
        ],
        out_specs=pl.BlockSpec((1, s, K), lambda i: (i, 0, 0)),
        out_shape=jax.ShapeDtypeStruct((b, s, K), jnp.int32),
    )(xyzt, new_xyzp)


def _sc_gather(data, indices):
    num = indices.shape[0]
    c = data.shape[1]
    window = 128
    mesh = plsc.VectorSubcoreMesh(core_axis_name="core",
                                  subcore_axis_name="subcore")
    idx2 = indices.reshape(1, num)

    @functools.partial(
        pl.kernel,
        out_type=jax.ShapeDtypeStruct((num, c), data.dtype),
        mesh=mesh)
    def gather_kernel(x_hbm, i_hbm, o_hbm):
        def body(i_vmem, o_vmem):
            pltpu.sync_copy(x_hbm.at[i_vmem.at[0]], o_vmem)

        pltpu.emit_pipeline(
            body,
            grid=(num // window,),
            in_specs=[pl.BlockSpec((1, window), index_map=lambda i: (0, i))],
            out_specs=[pl.BlockSpec((window, c), index_map=lambda i: (i, 0))],
            core_axis_name='subcore',
            dimension_semantics=(pltpu.PARALLEL,),
        )(i_hbm, o_hbm)

    return gather_kernel(data, idx2)


def _samlp_kernel(K, n_layers, g_ref, c_ref, *refs):
    wrefs, out_ref = refs[:2 * n_layers], refs[-1]
    x = g_ref[0]
    ctr = c_ref[0]
    rs = ctr.shape[0]
    cp = ctr.shape[1]
    x = (x.reshape(rs, K, cp) - ctr[:, None, :]).reshape(rs * K, cp)
    for i in range(n_layers):
        w, bias = wrefs[2 * i][...], wrefs[2 * i + 1][...]
        x = jnp.maximum(_dot_ref(x, w) + bias, 0.0)
    cout = x.shape[1]
    out_ref[0] = jnp.max(x.reshape(rs, K, cout), axis=1)


def _sa_mlp(gathered, cpad, K, params, rs):
    b, sk, cp = gathered.shape
    s = sk // K
    n_layers = len(params)
    w1 = jnp.pad(params[0][0], [(0, cp - params[0][0].shape[0]), (0, 0)])
    weights = [w1, params[0][1]]
    for w, bias in params[1:]:
        weights += [w, bias]
    cout = params[-1][0].shape[1]
    in_specs = [
        pl.BlockSpec((1, rs * K, cp), lambda i, j: (i, j, 0)),
        pl.BlockSpec((1, rs, cp), lambda i, j: (i, j, 0)),
    ]
    for wgt in weights:
        if wgt.ndim == 2:
            in_specs.append(pl.BlockSpec(wgt.shape, lambda i, j: (0, 0)))
        else:
            in_specs.append(pl.BlockSpec(wgt.shape, lambda i, j: (0,)))
    return pl.pallas_call(
        functools.partial(_samlp_kernel, K, n_layers),
        grid=(b, s // rs),
        in_specs=in_specs,
        out_specs=pl.BlockSpec((1, rs, cout), lambda i, j: (i, j, 0)),
        out_shape=jax.ShapeDtypeStruct((b, s, cout), jnp.float32),
    )(gathered, cpad, *weights)


def _set_abstraction(xyz, feats, S, radius, K, params):
    b, n, _ = xyz.shape
    cin = 3 + feats.shape[-1]
    cp = -(-cin // 128) * 128
    idx = _fps(xyz, S)
    new_xyz = _gather(xyz, idx)
    xyzt = jnp.swapaxes(_pad_last(xyz, 8), 1, 2)
    group_idx = _ball_query_global(_pad_last(new_xyz, 8), xyzt, radius, K)
    data = _pad_last(jnp.concatenate([xyz, feats], axis=-1), cp)
    g = _sc_gather(data.reshape(b * n, cp), group_idx.reshape(-1))
    g = g.reshape(b, S * K, cp)
    cpad = _pad_last(new_xyz, cp)
    rs = max(1, 2048 // K)
    pooled = _sa_mlp(g, cpad, K, params, rs)
    return new_xyz, pooled


def _sa_all_kernel(x_ref, f_ref, w1a_ref, w1b_ref, b1_ref, w2_ref, b2_ref,
                   w3_ref, b3_ref, out_ref):
    x3 = x_ref[0]
    f3 = f_ref[0]
    h = _dot_ref(x3, w1a_ref[...]) + _dot_ref(f3, w1b_ref[...]) + b1_ref[...]
    h = jnp.maximum(h, 0.0)
    h = jnp.maximum(_dot_ref(h, w2_ref[...]) + b2_ref[...], 0.0)
    h = jnp.maximum(_dot_ref(h, w3_ref[...]) + b3_ref[...], 0.0)
    out_ref[0] = jnp.max(h, axis=0, keepdims=True)


def _set_abstraction_all(xyz, feats, params):
    b, n, _ = xyz.shape
    c2 = feats.shape[-1]
    (w1, b1), (w2, b2), (w3, b3) = params
    w1a = jnp.pad(w1[:3], [(0, 5), (0, 0)])
    w1b = w1[3:]
    cout = w3.shape[1]
    f4 = pl.pallas_call(
        _sa_all_kernel,
        grid=(b,),
        in_specs=[
            pl.BlockSpec((1, n, 8), lambda i: (i, 0, 0)),
            pl.BlockSpec((1, n, c2), lambda i: (i, 0, 0)),
            pl.BlockSpec(w1a.shape, lambda i: (0, 0)),
            pl.BlockSpec(w1b.shape, lambda i: (0, 0)),
            pl.BlockSpec(b1.shape, lambda i: (0,)),
            pl.BlockSpec(w2.shape, lambda i: (0, 0)),
            pl.BlockSpec(b2.shape, lambda i: (0,)),
            pl.BlockSpec(w3.shape, lambda i: (0, 0)),
            pl.BlockSpec(b3.shape, lambda i: (0,)),
        ],
        out_specs=pl.BlockSpec((1, 1, cout), lambda i: (i, 0, 0)),
        out_shape=jax.ShapeDtypeStruct((b, 1, cout), jnp.float32),
    )(_pad_last(xyz, 8), feats, w1a, w1b, b1, w2, b2, w3, b3)
    return jnp.mean(xyz, axis=1, keepdims=True), f4


def _fp_kernel(n_layers, with_seg, x1_ref, qt_ref, f1_ref, f2_ref, *refs):
    nw = 3 + (n_layers - 1) * 2 + (6 if with_seg else 0)
    wrefs, outs = refs[:nw], refs[nw:]
    P = x1_ref[0]
    QT = qt_ref[0]
    f1 = f1_ref[0]
    f2 = f2_ref[0]
    R, s = P.shape[0], QT.shape[1]
    d2 = ((P[:, 0:1] - QT[0:1, :]) ** 2
          + (P[:, 1:2] - QT[1:2, :]) ** 2
          + (P[:, 2:3] - QT[2:3, :]) ** 2)
    iota = jax.lax.broadcasted_iota(jnp.int32, (R, s), 1)
    wmat = jnp.zeros((R, s), jnp.float32)
    d2w = d2
    for _ in range(3):
        m = jnp.min(d2w, axis=1, keepdims=True)
        idx = jnp.min(jnp.where(d2w == m, iota, s), axis=1, keepdims=True)
        onehot = iota == idx
        wmat = wmat + jnp.where(onehot, 1.0 / (m + 1e-8), 0.0)
        d2w = jnp.where(onehot, jnp.float32(3.4e38), d2w)
    wsum = jnp.sum(wmat, axis=1, keepdims=True)
    interp = jnp.dot(wmat, f2, preferred_element_type=jnp.float32, precision=jax.lax.Precision.HIGHEST) / wsum
    w1a, w1b, b1 = wrefs[0][...], wrefs[1][...], wrefs[2][...]
    h = _dot_ref(f1, w1a) + _dot_ref(interp, w1b) + b1
    h = jnp.maximum(h, 0.0)
    k = 3
    for _ in range(n_layers - 1):
        w, b = wrefs[k][...], wrefs[k + 1][...]
        k += 2
        h = jnp.maximum(_dot_ref(h, w) + b, 0.0)
    outs[0][0] = h
    if with_seg:
        sw1, sb1, sw2, sb2, sw3, sb3 = (r[...] for r in wrefs[k:k + 6])
        g = _dot_ref(h, sw1) + sb1
        g = jnp.where(g > 0, g, 0.2 * g)
        g = _dot_ref(g, sw2) + sb2
        g = jnp.where(g > 0, g, 0.2 * g)
        outs[1][0] = _dot_ref(g, sw3) + sb3


def _pad_last(a, to):
    c = a.shape[-1]
    if c == to:
        return a
    return jnp.pad(a, [(0, 0)] * (a.ndim - 1) + [(0, to - c)])


def _feature_propagation_fused(xyz1, xyz2, feats1, feats2, params, tile,
                               seg_params=None):
    b, n = xyz1.shape[0], xyz1.shape[1]
    s = xyz2.shape[1]
    c1 = feats1.shape[-1]
    c1p = 16 if c1 < 16 else c1
    c2 = feats2.shape[-1]
    x1p = _pad_last(xyz1, 8)
    qt = jnp.swapaxes(_pad_last(xyz2, 8), 1, 2)
    f1p = _pad_last(feats1, c1p)
    w1 = params[0][0]
    w1a, w1b = w1[:c1], w1[c1:]
    w1a = jnp.pad(w1a, [(0, c1p - c1), (0, 0)])
    weights = [w1a, w1b, params[0][1]]
    for w, bias in params[1:]:
        weights += [w, bias]
    n_layers = len(params)
    with_seg = seg_params is not None
    if with_seg:
        for w, bias in seg_params:
            weights += [w, bias]
    cout = params[-1][0].shape[1]
    grid = (b, n // tile)
    in_specs = [
        pl.BlockSpec((1, tile, 8), lambda i, j: (i, j, 0)),
        pl.BlockSpec((1, 8, s), lambda i, j: (i, 0, 0)),
        pl.BlockSpec((1, tile, c1p), lambda i, j: (i, j, 0)),
        pl.BlockSpec((1, s, c2), lambda i, j: (i, 0, 0)),
    ]
    for wgt in weights:
        if wgt.ndim == 2:
            in_specs.append(pl.BlockSpec(wgt.shape, lambda i, j: (0, 0)))
        else:
            in_specs.append(pl.BlockSpec(wgt.shape, lambda i, j: (0,)))
    out_specs = [pl.BlockSpec((1, tile, cout), lambda i, j: (i, j, 0))]
    out_shape = [jax.ShapeDtypeStruct((b, n, cout), jnp.float32)]
    if with_seg:
        l = seg_params[-1][0].shape[1]
        out_specs.append(pl.BlockSpec((1, tile, l), lambda i, j: (i, j, 0)))
        out_shape.append(jax.ShapeDtypeStruct((b, n, l), jnp.float32))
    outs = pl.pallas_call(
        functools.partial(_fp_kernel, n_layers, with_seg),
        grid=grid,
        in_specs=in_specs,
        out_specs=out_specs,
        out_shape=out_shape,
    )(x1p, qt, f1p, feats2, *weights)
    return outs if with_seg else outs[0]


def _fp3_kernel(f3_ref, f4_ref, w1a_ref, w1b_ref, b1_ref, w2_ref, b2_ref,
                out_ref):
    f3 = f3_ref[0]
    f4 = f4_ref[0]
    h = _dot_ref(f3, w1a_ref[...]) + _dot_ref(f4, w1b_ref[...]) + b1_ref[...]
    h = jnp.maximum(h, 0.0)
    h = _dot_ref(h, w2_ref[...]) + b2_ref[...]
    out_ref[0] = jnp.maximum(h, 0.0)


def _feature_propagation_bcast(feats1, feats2, params):
    b, n, c1 = feats1.shape
    c2 = feats2.shape[-1]
    (w1, b1), (w2, b2) = params
    w1a, w1b = w1[:c1], w1[c1:]
    cout = w2.shape[1]
    return pl.pallas_call(
        _fp3_kernel,
        grid=(b,),
        in_specs=[
            pl.BlockSpec((1, n, c1), lambda i: (i, 0, 0)),
            pl.BlockSpec((1, 1, c2), lambda i: (i, 0, 0)),
            pl.BlockSpec(w1a.shape, lambda i: (0, 0)),
            pl.BlockSpec(w1b.shape, lambda i: (0, 0)),
            pl.BlockSpec(b1.shape, lambda i: (0,)),
            pl.BlockSpec(w2.shape, lambda i: (0, 0)),
            pl.BlockSpec(b2.shape, lambda i: (0,)),
        ],
        out_specs=pl.BlockSpec((1, n, cout), lambda i: (i, 0, 0)),
        out_shape=jax.ShapeDtypeStruct((b, n, cout), jnp.float32),
    )(feats1, feats2.reshape(b, 1, c2), w1a, w1b, b1, w2, b2)


def kernel(pointcloud, params):
    xyz0, f0 = pointcloud[..., :3], pointcloud[..., 3:]
    xyz1, f1 = _set_abstraction(xyz0, f0, 1024, 0.1, 32, params['sa1'])
    xyz2, f2 = _set_abstraction(xyz1, f1, 256, 0.2, 64, params['sa2'])
    xyz3, f3 = _set_abstraction(xyz2, f2, 64, 0.4, 128, params['sa3'])
    xyz4, f4 = _set_abstraction_all(xyz3, f3, params['sa_all'])
    f3 = _feature_propagation_bcast(f3, f4, params['fp3'])
    f2 = _feature_propagation_fused(xyz2, xyz3, f2, f3, params['fp2'], 256)
    f1 = _feature_propagation_fused(xyz1, xyz2, f1, f2, params['fp1'], 512)
    point_features, logits = _feature_propagation_fused(
        xyz0, xyz1, pointcloud, f1, params['fp0'], 512,
        seg_params=params['seg'])
    global_features = f4.reshape(f4.shape[0], 512)
    return (point_features, global_features, logits)


# speedup vs baseline: 10.4128x; 1.2946x over previous
import functools

import jax
import jax.numpy as jnp
from jax.experimental import pallas as pl
from jax.experimental.pallas import tpu as pltpu
from jax.experimental.pallas import tpu_sc as plsc

_HI = jax.lax.Precision.HIGHEST


def _dot_ref(x, w):
    return jnp.dot(x.astype(jnp.bfloat16), w.astype(jnp.bfloat16),
                   preferred_element_type=jnp.float32)


def _sqdist(a, b):
    return jnp.sum((a[:, :, None, :] - b[:, None, :, :]) ** 2, axis=-1)


def _fps_kernel(S, x_ref, y_ref, z_ref, out_ref):
    b, n = x_ref.shape
    x = x_ref[...]
    y = y_ref[...]
    z = z_ref[...]
    iota_n = jax.lax.broadcasted_iota(jnp.int32, (b, n), 1)
    iota_s = jax.lax.broadcasted_iota(jnp.int32, (b, S), 1)

    def body(i, state):
        dists, sel, idx_col = state
        onehot = (iota_n == idx_col).astype(jnp.float32)
        lx = jnp.sum(x * onehot, axis=1, keepdims=True)
        ly = jnp.sum(y * onehot, axis=1, keepdims=True)
        lz = jnp.sum(z * onehot, axis=1, keepdims=True)
        d = (x - lx) ** 2 + (y - ly) ** 2 + (z - lz) ** 2
        dists = jnp.minimum(dists, d)
        m = jnp.max(dists, axis=1, keepdims=True)
        idx_col = jnp.min(jnp.where(dists == m, iota_n, n), axis=1,
                          keepdims=True)
        sel = jnp.where(iota_s == i, idx_col, sel)
        return (dists, sel, idx_col)

    dists0 = jnp.full((b, n), 1e10, dtype=jnp.float32)
    sel0 = jnp.zeros((b, S), dtype=jnp.int32)
    idx0 = jnp.zeros((b, 1), dtype=jnp.int32)
    _, sel, _ = jax.lax.fori_loop(1, S, body, (dists0, sel0, idx0))
    out_ref[...] = sel


def _fps(xyz, S):
    b, n, _ = xyz.shape
    x = xyz[..., 0]
    y = xyz[..., 1]
    z = xyz[..., 2]
    return pl.pallas_call(
        functools.partial(_fps_kernel, S),
        in_specs=[pl.BlockSpec((b, n), lambda: (0, 0))] * 3,
        out_specs=pl.BlockSpec((b, S), lambda: (0, 0)),
        out_shape=jax.ShapeDtypeStruct((b, S), jnp.int32),
    )(x, y, z)


def _gather(pts, idx):
    return jax.vmap(lambda p, i: p[i])(pts, idx)


def _ball_query(new_xyz, xyz, radius, K):
    n = xyz.shape[1]
    d2 = _sqdist(new_xyz, xyz)
    nn = jnp.argmin(d2, axis=-1).astype(jnp.int32)
    cand = jnp.where(d2 < radius * radius,
                     jnp.arange(n, dtype=jnp.int32)[None, None, :], n)
    cand = jnp.sort(cand, axis=-1)[..., :K]
    first = cand[..., :1]
    cand = jnp.where(cand == n, jnp.broadcast_to(first, cand.shape), cand)
    cand = jnp.where(cand == n, nn[..., None], cand)
    return cand


def _mlp(x, params):
    for W, b in params:
        x = jnp.maximum(x @ W + b, 0.0)
    return x


def _ballq_kernel(radius2, K, p_ref, c_ref, out_ref):
    pts = p_ref[0]
    ctr = c_ref[0]
    n = pts.shape[1]
    s = ctr.shape[0]
    nchunks = n // 128
    cx, cy, cz = ctr[:, 0:1], ctr[:, 1:2], ctr[:, 2:3]
    iota128 = jax.lax.broadcasted_iota(jnp.int32, (s, 128), 1)
    iota_k = jax.lax.broadcasted_iota(jnp.int32, (s, K), 1)
    folded = jnp.zeros((s, 128), jnp.int32)
    dmin = jnp.full((s, 1), 1e30, jnp.float32)
    nn = jnp.zeros((s, 1), jnp.int32)
    for q in range(nchunks):
        px = pts[0:1, q * 128:(q + 1) * 128]
        py = pts[1:2, q * 128:(q + 1) * 128]
        pz = pts[2:3, q * 128:(q + 1) * 128]
        d = (cx - px) ** 2 + (cy - py) ** 2 + (cz - pz) ** 2
        kept = d < radius2
        bit = jnp.int32(-2**31) if q == 31 else jnp.int32(1 << q)
        folded = folded | jnp.where(kept, bit, 0)
        cm = jnp.min(d, axis=1, keepdims=True)
        carg = jnp.min(jnp.where(d == cm, iota128 + q * 128, n), axis=1,
                       keepdims=True)
        nn = jnp.where(cm < dmin, carg, nn)
        dmin = jnp.minimum(dmin, cm)

    def body(k, state):
        folded, sel = state
        lsb = folded & (-folded)
        f = lsb.astype(jnp.float32)
        q = ((jax.lax.bitcast_convert_type(f, jnp.int32) >> 23) & 255) - 127
        jl = jnp.where(folded != 0, q * 128 + iota128, n)
        m = jnp.min(jl, axis=1, keepdims=True)
        sel = jnp.where(iota_k == k, m, sel)
        folded = jnp.where((iota128 == (m & 127)) & (m < n),
                           folded & (folded - 1), folded)
        return (folded, sel)

    sel0 = jnp.full((s, K), n, jnp.int32)
    _, sel = jax.lax.fori_loop(0, K, body, (folded, sel0))
    first = sel[:, 0:1]
    sel = jnp.where(sel == n, first, sel)
    sel = jnp.where(sel == n, nn, sel)
    out_ref[0] = sel + pl.program_id(0) * n


def _ball_query_global(new_xyzp, xyzt, radius, K):
    b, s, _ = new_xyzp.shape
    n = xyzt.shape[2]
    return pl.pallas_call(
        functools.partial(_ballq_kernel, radius * radius, K),
        grid=(b,),
        in_specs=[
            pl.BlockSpec((1, 8, n), lambda i: (i, 0, 0)),
            pl.BlockSpec((1, s, 8), lambda i: (i, 0, 0)),
        ],
        out_specs=pl.BlockSpec((1, s, K), lambda i: (i, 0, 0)),
        out_shape=jax.ShapeDtypeStruct((b, s, K), jnp.int32),
    )(xyzt, new_xyzp)


def _sc_gather(data, indices):
    num = indices.shape[0]
    c = data.shape[1]
    window = 128
    mesh = plsc.VectorSubcoreMesh(core_axis_name="core",
                                  subcore_axis_name="subcore")
    idx2 = indices.reshape(1, num)

    @functools.partial(
        pl.kernel,
        out_type=jax.ShapeDtypeStruct((num, c), data.dtype),
        mesh=mesh)
    def gather_kernel(x_hbm, i_hbm, o_hbm):
        def body(i_vmem, o_vmem):
            pltpu.sync_copy(x_hbm.at[i_vmem.at[0]], o_vmem)

        pltpu.emit_pipeline(
            body,
            grid=(num // window,),
            in_specs=[pl.BlockSpec((1, window), index_map=lambda i: (0, i))],
            out_specs=[pl.BlockSpec((window, c), index_map=lambda i: (i, 0))],
            core_axis_name=('core', 'subcore'),
            dimension_semantics=(pltpu.PARALLEL,),
        )(i_hbm, o_hbm)

    return gather_kernel(data, idx2)


def _samlp_kernel(K, n_layers, g_ref, c_ref, *refs):
    wrefs, out_ref = refs[:2 * n_layers], refs[-1]
    x = g_ref[0]
    ctr = c_ref[0]
    rs = ctr.shape[0]
    cp = ctr.shape[1]
    x = (x.reshape(rs, K, cp) - ctr[:, None, :]).reshape(rs * K, cp)
    for i in range(n_layers):
        w, bias = wrefs[2 * i][...], wrefs[2 * i + 1][...]
        x = jnp.maximum(_dot_ref(x, w) + bias, 0.0)
    cout = x.shape[1]
    out_ref[0] = jnp.max(x.reshape(rs, K, cout), axis=1)


def _sa_mlp(gathered, cpad, K, params, rs):
    b, sk, cp = gathered.shape
    s = sk // K
    n_layers = len(params)
    w1 = jnp.pad(params[0][0], [(0, cp - params[0][0].shape[0]), (0, 0)])
    weights = [w1, params[0][1]]
    for w, bias in params[1:]:
        weights += [w, bias]
    cout = params[-1][0].shape[1]
    in_specs = [
        pl.BlockSpec((1, rs * K, cp), lambda i, j: (i, j, 0)),
        pl.BlockSpec((1, rs, cp), lambda i, j: (i, j, 0)),
    ]
    for wgt in weights:
        if wgt.ndim == 2:
            in_specs.append(pl.BlockSpec(wgt.shape, lambda i, j: (0, 0)))
        else:
            in_specs.append(pl.BlockSpec(wgt.shape, lambda i, j: (0,)))
    return pl.pallas_call(
        functools.partial(_samlp_kernel, K, n_layers),
        grid=(b, s // rs),
        in_specs=in_specs,
        out_specs=pl.BlockSpec((1, rs, cout), lambda i, j: (i, j, 0)),
        out_shape=jax.ShapeDtypeStruct((b, s, cout), jnp.float32),
    )(gathered, cpad, *weights)


def _set_abstraction(xyz, feats, S, radius, K, params):
    b, n, _ = xyz.shape
    cin = 3 + feats.shape[-1]
    cp = -(-cin // 128) * 128
    idx = _fps(xyz, S)
    new_xyz = _gather(xyz, idx)
    xyzt = jnp.swapaxes(_pad_last(xyz, 8), 1, 2)
    group_idx = _ball_query_global(_pad_last(new_xyz, 8), xyzt, radius, K)
    data = _pad_last(jnp.concatenate([xyz, feats], axis=-1), cp)
    g = _sc_gather(data.reshape(b * n, cp), group_idx.reshape(-1))
    g = g.reshape(b, S * K, cp)
    cpad = _pad_last(new_xyz, cp)
    rs = max(1, 2048 // K)
    pooled = _sa_mlp(g, cpad, K, params, rs)
    return new_xyz, pooled


def _sa_all_kernel(x_ref, f_ref, w1a_ref, w1b_ref, b1_ref, w2_ref, b2_ref,
                   w3_ref, b3_ref, out_ref):
    x3 = x_ref[0]
    f3 = f_ref[0]
    h = _dot_ref(x3, w1a_ref[...]) + _dot_ref(f3, w1b_ref[...]) + b1_ref[...]
    h = jnp.maximum(h, 0.0)
    h = jnp.maximum(_dot_ref(h, w2_ref[...]) + b2_ref[...], 0.0)
    h = jnp.maximum(_dot_ref(h, w3_ref[...]) + b3_ref[...], 0.0)
    out_ref[0] = jnp.max(h, axis=0, keepdims=True)


def _set_abstraction_all(xyz, feats, params):
    b, n, _ = xyz.shape
    c2 = feats.shape[-1]
    (w1, b1), (w2, b2), (w3, b3) = params
    w1a = jnp.pad(w1[:3], [(0, 5), (0, 0)])
    w1b = w1[3:]
    cout = w3.shape[1]
    f4 = pl.pallas_call(
        _sa_all_kernel,
        grid=(b,),
        in_specs=[
            pl.BlockSpec((1, n, 8), lambda i: (i, 0, 0)),
            pl.BlockSpec((1, n, c2), lambda i: (i, 0, 0)),
            pl.BlockSpec(w1a.shape, lambda i: (0, 0)),
            pl.BlockSpec(w1b.shape, lambda i: (0, 0)),
            pl.BlockSpec(b1.shape, lambda i: (0,)),
            pl.BlockSpec(w2.shape, lambda i: (0, 0)),
            pl.BlockSpec(b2.shape, lambda i: (0,)),
            pl.BlockSpec(w3.shape, lambda i: (0, 0)),
            pl.BlockSpec(b3.shape, lambda i: (0,)),
        ],
        out_specs=pl.BlockSpec((1, 1, cout), lambda i: (i, 0, 0)),
        out_shape=jax.ShapeDtypeStruct((b, 1, cout), jnp.float32),
    )(_pad_last(xyz, 8), feats, w1a, w1b, b1, w2, b2, w3, b3)
    return jnp.mean(xyz, axis=1, keepdims=True), f4


def _fp_kernel(n_layers, with_seg, x1_ref, qt_ref, f1_ref, f2_ref, *refs):
    nw = 3 + (n_layers - 1) * 2 + (6 if with_seg else 0)
    wrefs, outs = refs[:nw], refs[nw:]
    P = x1_ref[0]
    QT = qt_ref[0]
    f1 = f1_ref[0]
    f2 = f2_ref[0]
    R, s = P.shape[0], QT.shape[1]
    d2 = ((P[:, 0:1] - QT[0:1, :]) ** 2
          + (P[:, 1:2] - QT[1:2, :]) ** 2
          + (P[:, 2:3] - QT[2:3, :]) ** 2)
    iota = jax.lax.broadcasted_iota(jnp.int32, (R, s), 1)
    wmat = jnp.zeros((R, s), jnp.float32)
    d2w = d2
    for _ in range(3):
        m = jnp.min(d2w, axis=1, keepdims=True)
        idx = jnp.min(jnp.where(d2w == m, iota, s), axis=1, keepdims=True)
        onehot = iota == idx
        wmat = wmat + jnp.where(onehot, 1.0 / (m + 1e-8), 0.0)
        d2w = jnp.where(onehot, jnp.float32(3.4e38), d2w)
    wsum = jnp.sum(wmat, axis=1, keepdims=True)
    interp = jnp.dot(wmat, f2, preferred_element_type=jnp.float32, precision=jax.lax.Precision.HIGHEST) / wsum
    w1a, w1b, b1 = wrefs[0][...], wrefs[1][...], wrefs[2][...]
    h = _dot_ref(f1, w1a) + _dot_ref(interp, w1b) + b1
    h = jnp.maximum(h, 0.0)
    k = 3
    for _ in range(n_layers - 1):
        w, b = wrefs[k][...], wrefs[k + 1][...]
        k += 2
        h = jnp.maximum(_dot_ref(h, w) + b, 0.0)
    outs[0][0] = h
    if with_seg:
        sw1, sb1, sw2, sb2, sw3, sb3 = (r[...] for r in wrefs[k:k + 6])
        g = _dot_ref(h, sw1) + sb1
        g = jnp.where(g > 0, g, 0.2 * g)
        g = _dot_ref(g, sw2) + sb2
        g = jnp.where(g > 0, g, 0.2 * g)
        outs[1][0] = _dot_ref(g, sw3) + sb3


def _pad_last(a, to):
    c = a.shape[-1]
    if c == to:
        return a
    return jnp.pad(a, [(0, 0)] * (a.ndim - 1) + [(0, to - c)])


def _feature_propagation_fused(xyz1, xyz2, feats1, feats2, params, tile,
                               seg_params=None):
    b, n = xyz1.shape[0], xyz1.shape[1]
    s = xyz2.shape[1]
    c1 = feats1.shape[-1]
    c1p = 16 if c1 < 16 else c1
    c2 = feats2.shape[-1]
    x1p = _pad_last(xyz1, 8)
    qt = jnp.swapaxes(_pad_last(xyz2, 8), 1, 2)
    f1p = _pad_last(feats1, c1p)
    w1 = params[0][0]
    w1a, w1b = w1[:c1], w1[c1:]
    w1a = jnp.pad(w1a, [(0, c1p - c1), (0, 0)])
    weights = [w1a, w1b, params[0][1]]
    for w, bias in params[1:]:
        weights += [w, bias]
    n_layers = len(params)
    with_seg = seg_params is not None
    if with_seg:
        for w, bias in seg_params:
            weights += [w, bias]
    cout = params[-1][0].shape[1]
    grid = (b, n // tile)
    in_specs = [
        pl.BlockSpec((1, tile, 8), lambda i, j: (i, j, 0)),
        pl.BlockSpec((1, 8, s), lambda i, j: (i, 0, 0)),
        pl.BlockSpec((1, tile, c1p), lambda i, j: (i, j, 0)),
        pl.BlockSpec((1, s, c2), lambda i, j: (i, 0, 0)),
    ]
    for wgt in weights:
        if wgt.ndim == 2:
            in_specs.append(pl.BlockSpec(wgt.shape, lambda i, j: (0, 0)))
        else:
            in_specs.append(pl.BlockSpec(wgt.shape, lambda i, j: (0,)))
    out_specs = [pl.BlockSpec((1, tile, cout), lambda i, j: (i, j, 0))]
    out_shape = [jax.ShapeDtypeStruct((b, n, cout), jnp.float32)]
    if with_seg:
        l = seg_params[-1][0].shape[1]
        out_specs.append(pl.BlockSpec((1, tile, l), lambda i, j: (i, j, 0)))
        out_shape.append(jax.ShapeDtypeStruct((b, n, l), jnp.float32))
    outs = pl.pallas_call(
        functools.partial(_fp_kernel, n_layers, with_seg),
        grid=grid,
        in_specs=in_specs,
        out_specs=out_specs,
        out_shape=out_shape,
    )(x1p, qt, f1p, feats2, *weights)
    return outs if with_seg else outs[0]


def _fp3_kernel(f3_ref, f4_ref, w1a_ref, w1b_ref, b1_ref, w2_ref, b2_ref,
                out_ref):
    f3 = f3_ref[0]
    f4 = f4_ref[0]
    h = _dot_ref(f3, w1a_ref[...]) + _dot_ref(f4, w1b_ref[...]) + b1_ref[...]
    h = jnp.maximum(h, 0.0)
    h = _dot_ref(h, w2_ref[...]) + b2_ref[...]
    out_ref[0] = jnp.maximum(h, 0.0)


def _feature_propagation_bcast(feats1, feats2, params):
    b, n, c1 = feats1.shape
    c2 = feats2.shape[-1]
    (w1, b1), (w2, b2) = params
    w1a, w1b = w1[:c1], w1[c1:]
    cout = w2.shape[1]
    return pl.pallas_call(
        _fp3_kernel,
        grid=(b,),
        in_specs=[
            pl.BlockSpec((1, n, c1), lambda i: (i, 0, 0)),
            pl.BlockSpec((1, 1, c2), lambda i: (i, 0, 0)),
            pl.BlockSpec(w1a.shape, lambda i: (0, 0)),
            pl.BlockSpec(w1b.shape, lambda i: (0, 0)),
            pl.BlockSpec(b1.shape, lambda i: (0,)),
            pl.BlockSpec(w2.shape, lambda i: (0, 0)),
            pl.BlockSpec(b2.shape, lambda i: (0,)),
        ],
        out_specs=pl.BlockSpec((1, n, cout), lambda i: (i, 0, 0)),
        out_shape=jax.ShapeDtypeStruct((b, n, cout), jnp.float32),
    )(feats1, feats2.reshape(b, 1, c2), w1a, w1b, b1, w2, b2)


def kernel(pointcloud, params):
    xyz0, f0 = pointcloud[..., :3], pointcloud[..., 3:]
    xyz1, f1 = _set_abstraction(xyz0, f0, 1024, 0.1, 32, params['sa1'])
    xyz2, f2 = _set_abstraction(xyz1, f1, 256, 0.2, 64, params['sa2'])
    xyz3, f3 = _set_abstraction(xyz2, f2, 64, 0.4, 128, params['sa3'])
    xyz4, f4 = _set_abstraction_all(xyz3, f3, params['sa_all'])
    f3 = _feature_propagation_bcast(f3, f4, params['fp3'])
    f2 = _feature_propagation_fused(xyz2, xyz3, f2, f3, params['fp2'], 256)
    f1 = _feature_propagation_fused(xyz1, xyz2, f1, f2, params['fp1'], 512)
    point_features, logits = _feature_propagation_fused(
        xyz0, xyz1, pointcloud, f1, params['fp0'], 512,
        seg_params=params['seg'])
    global_features = f4.reshape(f4.shape[0], 512)
    return (point_features, global_features, logits)
